# baseline (speedup 1.0000x reference)
import jax
import jax.numpy as jnp
from jax.experimental import pallas as pl
from jax.experimental.pallas import tpu as pltpu

NUM_EMBEDDINGS = 8192
EMBEDDING_DIM = 32
BETA = 0.25

T_TILE = 256
K_CHUNK = 2048
N_K = NUM_EMBEDDINGS // K_CHUNK
D2 = 2 * EMBEDDING_DIM


def _vq_tile_kernel(x_ref, e_ref, q_ref, loss_ref, es_ref, esq_ref):
    @pl.when(pl.program_id(0) == 0)
    def _init():
        e = e_ref[...]
        e_hi = e.astype(jnp.bfloat16)
        e_lo = (e - e_hi.astype(jnp.float32)).astype(jnp.bfloat16)
        es_ref[0:EMBEDDING_DIM, :] = e_hi
        es_ref[EMBEDDING_DIM:D2, :] = e_lo
        es_ref[D2:D2 + 8, :] = jnp.concatenate(
            [jnp.ones((1, NUM_EMBEDDINGS), jnp.bfloat16),
             jnp.zeros((7, NUM_EMBEDDINGS), jnp.bfloat16)], axis=0)
        esq_ref[...] = jnp.sum(e * e, axis=0, keepdims=True)

    xt = x_ref[...]
    xb2 = (xt + xt).astype(jnp.bfloat16)
    xsq = jnp.sum(xt * xt, axis=1)

    def chunk_d(c):
        sl = pl.ds(c * K_CHUNK, K_CHUNK)
        sim2 = jax.lax.dot_general(
            xb2, es_ref[0:EMBEDDING_DIM, sl], (((1,), (0,)), ((), ())),
            preferred_element_type=jnp.float32)
        d = (xsq[:, None] + esq_ref[0, sl][None, :]) - sim2
        return d, jnp.min(d, axis=1)

    def gather(onehot, c):
        sl = pl.ds(c * K_CHUNK, K_CHUNK)
        q_2 = jax.lax.dot_general(
            onehot, es_ref[:, sl], (((1,), (1,)), ((), ())),
            preferred_element_type=jnp.float32)
        return q_2[:, :EMBEDDING_DIM] + q_2[:, EMBEDDING_DIM:D2], q_2[:, D2]

    cur = jnp.full((T_TILE,), jnp.inf, dtype=jnp.float32)
    dwin = jnp.zeros((T_TILE,), dtype=jnp.float32)
    q = jnp.zeros((T_TILE, EMBEDDING_DIM), dtype=jnp.float32)
    mx = jnp.float32(0.0)
    for c in range(N_K):
        d, cm = chunk_d(c)
        onehot = (d == cm[:, None]).astype(jnp.bfloat16)
        q_c, cnt = gather(onehot, c)
        mx = jnp.maximum(mx, jnp.max(cnt))
        upd = cm < cur
        q = jnp.where(upd[:, None], q_c, q)
        dwin = jnp.where(upd, cm, dwin)
        cur = jnp.where(upd, cm, cur).astype(jnp.bfloat16).astype(jnp.float32)
    q_ref[...] = q
    loss_ref[...] = jnp.broadcast_to(jnp.sum(dwin), (1, 1, 128))

    @pl.when(mx > 1.5)
    def _tie_fallback():
        lane_iota = jax.lax.broadcasted_iota(jnp.int32, (T_TILE, K_CHUNK), 1)
        cur2 = jnp.full((T_TILE,), jnp.inf, dtype=jnp.float32)
        q2 = jnp.zeros((T_TILE, EMBEDDING_DIM), dtype=jnp.float32)
        for c in range(N_K):
            d, cm = chunk_d(c)
            idx = jnp.min(
                jnp.where(d == cm[:, None], lane_iota, K_CHUNK), axis=1)
            oh1 = (lane_iota == idx[:, None]).astype(jnp.bfloat16)
            q_c, _ = gather(oh1, c)
            upd = cm < cur2
            q2 = jnp.where(upd[:, None], q_c, q2)
            cur2 = jnp.where(upd, cm, cur2).astype(
                jnp.bfloat16).astype(jnp.float32)
        q_ref[...] = q2


@jax.jit
def kernel(x, embeddings):
    input_shape = x.shape
    xf = x.reshape(-1, EMBEDDING_DIM)
    n_t = xf.shape[0] // T_TILE
    q, loss_part = pl.pallas_call(
        _vq_tile_kernel,
        grid=(n_t,),
        in_specs=[
            pl.BlockSpec((T_TILE, EMBEDDING_DIM), lambda i: (i, 0)),
            pl.BlockSpec((EMBEDDING_DIM, NUM_EMBEDDINGS), lambda i: (0, 0)),
        ],
        out_specs=[
            pl.BlockSpec((T_TILE, EMBEDDING_DIM), lambda i: (i, 0)),
            pl.BlockSpec((1, 1, 128), lambda i: (i, 0, 0)),
        ],
        out_shape=[
            jax.ShapeDtypeStruct((xf.shape[0], EMBEDDING_DIM), jnp.float32),
            jax.ShapeDtypeStruct((n_t, 1, 128), jnp.float32),
        ],
        scratch_shapes=[
            pltpu.VMEM((D2 + 8, NUM_EMBEDDINGS), jnp.bfloat16),
            pltpu.VMEM((1, NUM_EMBEDDINGS), jnp.float32),
        ],
    )(xf, embeddings)
    total = jnp.sum(loss_part[:, 0, 0])
    loss = (1.0 + BETA) * total / xf.size
    return q.reshape(input_shape), loss


# speedup vs baseline: 1.0102x; 1.0102x over previous
import jax
import jax.numpy as jnp
from jax.experimental import pallas as pl
from jax.experimental.pallas import tpu as pltpu

NUM_EMBEDDINGS = 8192
EMBEDDING_DIM = 32
BETA = 0.25

T_TILE = 512
K_CHUNK = 2048
N_K = NUM_EMBEDDINGS // K_CHUNK
D2 = 2 * EMBEDDING_DIM


def _vq_tile_kernel(x_ref, e_ref, q_ref, loss_ref, es_ref, esq_ref):
    @pl.when(pl.program_id(0) == 0)
    def _init():
        e = e_ref[...]
        e_hi = e.astype(jnp.bfloat16)
        e_lo = (e - e_hi.astype(jnp.float32)).astype(jnp.bfloat16)
        es_ref[0:EMBEDDING_DIM, :] = e_hi
        es_ref[EMBEDDING_DIM:D2, :] = e_lo
        es_ref[D2:D2 + 8, :] = jnp.concatenate(
            [jnp.ones((1, NUM_EMBEDDINGS), jnp.bfloat16),
             jnp.zeros((7, NUM_EMBEDDINGS), jnp.bfloat16)], axis=0)
        esq_ref[...] = jnp.sum(e * e, axis=0, keepdims=True)

    xt = x_ref[...]
    xb2 = (xt + xt).astype(jnp.bfloat16)
    xsq = jnp.sum(xt * xt, axis=1)

    def chunk_d(c):
        sl = pl.ds(c * K_CHUNK, K_CHUNK)
        sim2 = jax.lax.dot_general(
            xb2, es_ref[0:EMBEDDING_DIM, sl], (((1,), (0,)), ((), ())),
            preferred_element_type=jnp.float32)
        d = (xsq[:, None] + esq_ref[0, sl][None, :]) - sim2
        return d, jnp.min(d, axis=1)

    def gather(onehot, c):
        sl = pl.ds(c * K_CHUNK, K_CHUNK)
        q_2 = jax.lax.dot_general(
            onehot, es_ref[:, sl], (((1,), (1,)), ((), ())),
            preferred_element_type=jnp.float32)
        return q_2[:, :EMBEDDING_DIM] + q_2[:, EMBEDDING_DIM:D2], q_2[:, D2]

    cur = jnp.full((T_TILE,), jnp.inf, dtype=jnp.float32)
    dwin = jnp.zeros((T_TILE,), dtype=jnp.float32)
    q = jnp.zeros((T_TILE, EMBEDDING_DIM), dtype=jnp.float32)
    mx = jnp.float32(0.0)
    for c in range(N_K):
        d, cm = chunk_d(c)
        onehot = (d == cm[:, None]).astype(jnp.bfloat16)
        q_c, cnt = gather(onehot, c)
        mx = jnp.maximum(mx, jnp.max(cnt))
        upd = cm < cur
        q = jnp.where(upd[:, None], q_c, q)
        dwin = jnp.where(upd, cm, dwin)
        cur = jnp.where(upd, cm, cur).astype(jnp.bfloat16).astype(jnp.float32)
    q_ref[...] = q
    loss_ref[...] = jnp.broadcast_to(jnp.sum(dwin), (1, 1, 128))

    @pl.when(mx > 1.5)
    def _tie_fallback():
        lane_iota = jax.lax.broadcasted_iota(jnp.int32, (T_TILE, K_CHUNK), 1)
        cur2 = jnp.full((T_TILE,), jnp.inf, dtype=jnp.float32)
        q2 = jnp.zeros((T_TILE, EMBEDDING_DIM), dtype=jnp.float32)
        for c in range(N_K):
            d, cm = chunk_d(c)
            idx = jnp.min(
                jnp.where(d == cm[:, None], lane_iota, K_CHUNK), axis=1)
            oh1 = (lane_iota == idx[:, None]).astype(jnp.bfloat16)
            q_c, _ = gather(oh1, c)
            upd = cm < cur2
            q2 = jnp.where(upd[:, None], q_c, q2)
            cur2 = jnp.where(upd, cm, cur2).astype(
                jnp.bfloat16).astype(jnp.float32)
        q_ref[...] = q2


@jax.jit
def kernel(x, embeddings):
    input_shape = x.shape
    xf = x.reshape(-1, EMBEDDING_DIM)
    n_t = xf.shape[0] // T_TILE
    q, loss_part = pl.pallas_call(
        _vq_tile_kernel,
        grid=(n_t,),
        in_specs=[
            pl.BlockSpec((T_TILE, EMBEDDING_DIM), lambda i: (i, 0)),
            pl.BlockSpec((EMBEDDING_DIM, NUM_EMBEDDINGS), lambda i: (0, 0)),
        ],
        out_specs=[
            pl.BlockSpec((T_TILE, EMBEDDING_DIM), lambda i: (i, 0)),
            pl.BlockSpec((1, 1, 128), lambda i: (i, 0, 0)),
        ],
        out_shape=[
            jax.ShapeDtypeStruct((xf.shape[0], EMBEDDING_DIM), jnp.float32),
            jax.ShapeDtypeStruct((n_t, 1, 128), jnp.float32),
        ],
        scratch_shapes=[
            pltpu.VMEM((D2 + 8, NUM_EMBEDDINGS), jnp.bfloat16),
            pltpu.VMEM((1, NUM_EMBEDDINGS), jnp.float32),
        ],
    )(xf, embeddings)
    total = jnp.sum(loss_part[:, 0, 0])
    loss = (1.0 + BETA) * total / xf.size
    return q.reshape(input_shape), loss
